# baseline (speedup 1.0000x reference)
import jax
import jax.numpy as jnp
from jax import lax
from jax.experimental import pallas as pl
from jax.experimental.pallas import tpu as pltpu
from jax.experimental.pallas import tpu_sc as plsc

N = 10000
NP = 10240
E = 320000
D = 128
DE = 16

NC = 2
NS = 16
NW = NC * NS
EPW = E // NW
CH = 80
NGRP = 5
GRP = 25
RPT = NP // NS

_mesh = plsc.VectorSubcoreMesh(core_axis_name="c", subcore_axis_name="s")
_sc_params = pltpu.CompilerParams(use_tc_tiling_on_sc=False)
_sc_params_nl = pltpu.CompilerParams(use_tc_tiling_on_sc=False, needs_layout_passes=False)


def _sc_spmm_body(h_hbm, src_hbm, dst_hbm, z128_hbm,
                  outS,
                  idx_s, idx_d, rows, accS, semg, sems):
    cid = lax.axis_index("c")
    sid = lax.axis_index("s")
    wid = sid * NC + cid
    r0 = sid * RPT
    pltpu.sync_copy(z128_hbm, rows.at[0])

    def zinit(k, carry):
        pltpu.sync_copy(rows.at[0], accS.at[pl.ds(r0 + k * CH, CH)])
        return carry

    lax.fori_loop(0, RPT // CH, zinit, 0)
    plsc.subcore_barrier()

    def group(g, carry):
        pltpu.async_copy(src_hbm.at[wid, g], idx_s, semg.at[0])
        pltpu.async_copy(dst_hbm.at[wid, g], idx_d, semg.at[1])
        pltpu.make_async_copy(src_hbm.at[wid, g], idx_s, semg.at[0]).wait()
        pltpu.make_async_copy(dst_hbm.at[wid, g], idx_d, semg.at[1]).wait()
        pltpu.async_copy(h_hbm.at[idx_s.at[0]], rows.at[0], semg.at[0])
        pltpu.async_copy(h_hbm.at[idx_s.at[1]], rows.at[1], semg.at[1])

        def chunk(j, c2):
            b = lax.rem(j, 2)
            pltpu.make_async_copy(h_hbm.at[idx_s.at[j]], rows.at[b], semg.at[b]).wait()
            pltpu.async_copy(rows.at[b], accS.at[idx_d.at[j]], sems.at[b], add=True)

            @pl.when(j + 2 < GRP)
            def _():
                pltpu.make_async_copy(rows.at[b], accS.at[idx_d.at[j]], sems.at[b]).wait()
                pltpu.async_copy(h_hbm.at[idx_s.at[j + 2]], rows.at[b], semg.at[b])

            return c2

        lax.fori_loop(0, GRP, chunk, carry)
        pltpu.make_async_copy(rows.at[0], accS.at[idx_d.at[0]], sems.at[0]).wait()
        pltpu.make_async_copy(rows.at[1], accS.at[idx_d.at[1]], sems.at[1]).wait()
        return carry

    lax.fori_loop(0, NGRP, group, 0)
    plsc.subcore_barrier()
    pltpu.sync_copy(accS.at[pl.ds(r0, RPT)], outS.at[cid, pl.ds(r0, RPT)])


_sc_spmm = pl.kernel(
    _sc_spmm_body,
    out_type=jax.ShapeDtypeStruct((NC, NP, D), jnp.float32),
    mesh=_mesh,
    compiler_params=_sc_params,
    scratch_types=[
        pltpu.VMEM((GRP, CH), jnp.int32),
        pltpu.VMEM((GRP, CH), jnp.int32),
        pltpu.VMEM((2, CH, D), jnp.float32),
        pltpu.VMEM_SHARED((NP, D), jnp.float32),
        pltpu.SemaphoreType.DMA((2,)),
        pltpu.SemaphoreType.DMA((2,)),
    ],
)


CHE = 125
GRP2 = 16
NGRP2 = 5


def _sc_edgestat_body(dst_hbm, ef_hbm, z16_hbm, ones_hbm,
                      outE, outD,
                      idx_d, efg, onesv, accE, accD, semE, semD):
    cid = lax.axis_index("c")
    sid = lax.axis_index("s")
    wid = sid * NC + cid
    r0 = sid * RPT
    pltpu.sync_copy(z16_hbm.at[pl.ds(r0, RPT)], accE.at[pl.ds(r0, RPT)])
    pltpu.sync_copy(z16_hbm.at[pl.ds(r0, RPT)], accD.at[pl.ds(r0, RPT)])
    pltpu.sync_copy(ones_hbm, onesv)
    plsc.subcore_barrier()

    def group(g, carry):
        pltpu.async_copy(dst_hbm.at[wid, g], idx_d, semE)
        pltpu.async_copy(ef_hbm.at[wid, g], efg, semD)
        pltpu.make_async_copy(dst_hbm.at[wid, g], idx_d, semE).wait()
        pltpu.make_async_copy(ef_hbm.at[wid, g], efg, semD).wait()

        def chunk(j, c2):
            pltpu.async_copy(efg.at[j], accE.at[idx_d.at[j]], semE, add=True)
            pltpu.async_copy(onesv, accD.at[idx_d.at[j]], semD, add=True)
            return c2

        lax.fori_loop(0, GRP2, chunk, carry)

        def drain(j, c2):
            pltpu.make_async_copy(efg.at[0], accE.at[idx_d.at[0]], semE).wait()
            pltpu.make_async_copy(onesv, accD.at[idx_d.at[0]], semD).wait()
            return c2

        lax.fori_loop(0, GRP2, drain, carry)
        return carry

    lax.fori_loop(0, NGRP2, group, 0)
    plsc.subcore_barrier()
    pltpu.sync_copy(accE.at[pl.ds(r0, RPT)], outE.at[cid, pl.ds(r0, RPT)])
    pltpu.sync_copy(accD.at[pl.ds(r0, RPT)], outD.at[cid, pl.ds(r0, RPT)])


_sc_edgestat = pl.kernel(
    _sc_edgestat_body,
    out_type=[
        jax.ShapeDtypeStruct((NC, NP, DE), jnp.float32),
        jax.ShapeDtypeStruct((NC, NP, DE), jnp.float32),
    ],
    mesh=_mesh,
    compiler_params=_sc_params,
    scratch_types=[
        pltpu.VMEM((GRP2, CHE), jnp.int32),
        pltpu.VMEM((GRP2, CHE, DE), jnp.float32),
        pltpu.VMEM((CHE, DE), jnp.float32),
        pltpu.VMEM_SHARED((NP, DE), jnp.float32),
        pltpu.VMEM_SHARED((NP, DE), jnp.float32),
        pltpu.SemaphoreType.DMA,
        pltpu.SemaphoreType.DMA,
    ],
)


def _sc_edge_body(s_hbm, src_hbm, dst_hbm,
                  out_hbm,
                  sv_v, dv_v, s_v, o01, sem3):
    cid = lax.axis_index("c")
    sid = lax.axis_index("s")
    wid = sid * NC + cid
    e0 = wid * EPW
    pltpu.async_copy(s_hbm, s_v, sem3.at[0])
    pltpu.async_copy(src_hbm.at[pl.ds(e0, EPW)], sv_v, sem3.at[1])
    pltpu.async_copy(dst_hbm.at[pl.ds(e0, EPW)], dv_v, sem3.at[2])
    pltpu.make_async_copy(s_hbm, s_v, sem3.at[0]).wait()
    pltpu.make_async_copy(src_hbm.at[pl.ds(e0, EPW)], sv_v, sem3.at[1]).wait()
    pltpu.make_async_copy(dst_hbm.at[pl.ds(e0, EPW)], dv_v, sem3.at[2]).wait()

    lane = lax.iota(jnp.int32, 16)
    half = lax.shift_right_logical(lane, 1)
    par = lax.rem(lane, 2)

    def it(i, carry):
        kk = half + i * 8
        sv8 = plsc.load_gather(sv_v, [kk])
        dv8 = plsc.load_gather(dv_v, [kk])
        a = plsc.load_gather(s_v, [sv8 * 4 + par])
        b = plsc.load_gather(s_v, [dv8 * 4 + 2 + par])
        o01[pl.ds(i * 16, 16)] = a + b
        return carry

    lax.fori_loop(0, (2 * EPW) // 16, it, 0)
    pltpu.sync_copy(o01, out_hbm.at[pl.ds(2 * e0, 2 * EPW)])


_sc_edge = pl.kernel(
    _sc_edge_body,
    out_type=jax.ShapeDtypeStruct((2 * E,), jnp.float32),
    mesh=_mesh,
    compiler_params=_sc_params_nl,
    scratch_types=[
        pltpu.VMEM((EPW,), jnp.int32),
        pltpu.VMEM((EPW,), jnp.int32),
        pltpu.VMEM((N * 4,), jnp.float32),
        pltpu.VMEM((2 * EPW,), jnp.float32),
        pltpu.SemaphoreType.DMA((3,)),
    ],
)


BT = 2000
GRID = N // BT


def _tc_layer1_body(S2, h, E2, D2, Wm, bm, Wa, ba, out):
    Sv = S2[0] + S2[1]
    Ea = E2[0] + E2[1]
    deg = D2[0][:, 0:1] + D2[1][:, 0:1]
    hn = jnp.dot(Sv, Wm[0:D], preferred_element_type=jnp.float32)
    hn += jnp.dot(Ea, Wm[D:D + DE], preferred_element_type=jnp.float32)
    hn += deg * bm[...]
    hn /= jnp.maximum(deg, 1.0)
    o = jnp.dot(h[...], Wa[0:D], preferred_element_type=jnp.float32)
    o += jnp.dot(hn, Wa[D:2 * D], preferred_element_type=jnp.float32)
    out[...] = jnp.maximum(o + ba[...], 0.0)


def _tc_layer2_body(S2, h, E2, D2, Wm, bm, Wa, ba, W4, b4, outa):
    Sv = S2[0] + S2[1]
    Ea = E2[0] + E2[1]
    deg = D2[0][:, 0:1] + D2[1][:, 0:1]
    hn = jnp.dot(Sv, Wm[0:D], preferred_element_type=jnp.float32)
    hn += jnp.dot(Ea, Wm[D:D + DE], preferred_element_type=jnp.float32)
    hn += deg * bm[...]
    hn /= jnp.maximum(deg, 1.0)
    o = jnp.dot(h[...], Wa[0:D], preferred_element_type=jnp.float32)
    o += jnp.dot(hn, Wa[D:2 * D], preferred_element_type=jnp.float32)
    h2 = jnp.maximum(o + ba[...], 0.0)
    outa[...] = jnp.dot(h2, W4[...], preferred_element_type=jnp.float32) + b4[...]


def _row_spec(shape):
    nd = len(shape)
    if nd == 3:
        return pl.BlockSpec((shape[0], BT, shape[2]), lambda i: (0, i, 0))
    return pl.BlockSpec((BT, shape[1]), lambda i: (i, 0))


def _full_spec(shape):
    return pl.BlockSpec(shape, lambda i: tuple(0 for _ in shape))


def _tc_layer1(S2, h, E2, D2, Wm, bm, Wa, ba):
    return pl.pallas_call(
        _tc_layer1_body,
        grid=(GRID,),
        in_specs=[
            _row_spec((NC, NP, D)), _row_spec((N, D)),
            _row_spec((NC, NP, DE)), _row_spec((NC, NP, DE)),
            _full_spec((D + DE, D)), _full_spec((1, D)),
            _full_spec((2 * D, D)), _full_spec((1, D)),
        ],
        out_specs=_row_spec((N, D)),
        out_shape=jax.ShapeDtypeStruct((N, D), jnp.float32),
    )(S2, h, E2, D2, Wm, bm, Wa, ba)


def _tc_layer2(S2, h, E2, D2, Wm, bm, Wa, ba, W4, b4):
    return pl.pallas_call(
        _tc_layer2_body,
        grid=(GRID,),
        in_specs=[
            _row_spec((NC, NP, D)), _row_spec((N, D)),
            _row_spec((NC, NP, DE)), _row_spec((NC, NP, DE)),
            _full_spec((D + DE, D)), _full_spec((1, D)),
            _full_spec((2 * D, D)), _full_spec((1, D)),
            _full_spec((D, 4)), _full_spec((1, 4)),
        ],
        out_specs=_row_spec((N, 4)),
        out_shape=jax.ShapeDtypeStruct((N, 4), jnp.float32),
    )(S2, h, E2, D2, Wm, bm, Wa, ba, W4, b4)


def kernel(nfeats, efeats, edge_index, Wm1, bm1, Wa1, ba1, Wm2, bm2, Wa2, ba2, Wp, bp):
    h0 = nfeats[:, 0, :]
    ef = efeats[:, 0, :]
    src = edge_index[0]
    dst = edge_index[1]
    src4d = src.reshape(NW, NGRP, GRP, CH)
    dst4d = dst.reshape(NW, NGRP, GRP, CH)
    z128 = jnp.zeros((CH, D), jnp.float32)
    z16 = jnp.zeros((NP, DE), jnp.float32)
    ones = jnp.zeros((CH, DE), jnp.float32).at[:, 0].set(1.0)

    dst5d = dst.reshape(NW, NGRP2, GRP2, CHE)
    ef5d = ef.reshape(NW, NGRP2, GRP2, CHE, DE)
    ones125 = jnp.zeros((CHE, DE), jnp.float32).at[:, 0].set(1.0)
    E2, D2 = _sc_edgestat(dst5d, ef5d, z16, ones125)
    S2 = _sc_spmm(h0, src4d, dst4d, z128)
    h1 = _tc_layer1(S2, h0, E2, D2, Wm1, bm1.reshape(1, D), Wa1, ba1.reshape(1, D))
    S2b = _sc_spmm(h1, src4d, dst4d, z128)
    W4 = jnp.concatenate([Wp[:D], Wp[D:]], axis=1)
    b4 = (jnp.concatenate([bp, bp]) * 0.5).reshape(1, 4)
    s4 = _tc_layer2(S2b, h1, E2, D2, Wm2, bm2.reshape(1, D), Wa2, ba2.reshape(1, D),
                    W4, b4)
    oflat = _sc_edge(s4.reshape(N * 4), src, dst)
    return oflat.reshape(E, 2)


# speedup vs baseline: 1.4630x; 1.4630x over previous
import jax
import jax.numpy as jnp
from jax import lax
from jax.experimental import pallas as pl
from jax.experimental.pallas import tpu as pltpu
from jax.experimental.pallas import tpu_sc as plsc

N = 10000
NP = 10240
E = 320000
D = 128
DE = 16

NC = 2
NS = 16
NW = NC * NS
EPW = E // NW
CH = 80
NGRP = 5
GRP = 25
RPT = NP // NS

_mesh = plsc.VectorSubcoreMesh(core_axis_name="c", subcore_axis_name="s")
_sc_params = pltpu.CompilerParams(use_tc_tiling_on_sc=False)
_sc_params_nl = pltpu.CompilerParams(use_tc_tiling_on_sc=False, needs_layout_passes=False)


def _sc_spmm_body(h_hbm, src_hbm, dst_hbm, z128_hbm,
                  outS,
                  idx_s, idx_d, rows, accS, semg, sems):
    cid = lax.axis_index("c")
    sid = lax.axis_index("s")
    wid = sid * NC + cid
    r0 = sid * RPT
    pltpu.sync_copy(z128_hbm, rows.at[0])

    def zinit(k, carry):
        pltpu.sync_copy(rows.at[0], accS.at[pl.ds(r0 + k * CH, CH)])
        return carry

    lax.fori_loop(0, RPT // CH, zinit, 0)
    plsc.subcore_barrier()

    def group(g, carry):
        pltpu.async_copy(src_hbm.at[wid, g], idx_s, semg.at[0])
        pltpu.async_copy(dst_hbm.at[wid, g], idx_d, semg.at[1])
        pltpu.make_async_copy(src_hbm.at[wid, g], idx_s, semg.at[0]).wait()
        pltpu.make_async_copy(dst_hbm.at[wid, g], idx_d, semg.at[1]).wait()
        pltpu.async_copy(h_hbm.at[idx_s.at[0]], rows.at[0], semg.at[0])
        pltpu.async_copy(h_hbm.at[idx_s.at[1]], rows.at[1], semg.at[1])

        def chunk(j, c2):
            b = lax.rem(j, 2)
            pltpu.make_async_copy(h_hbm.at[idx_s.at[j]], rows.at[b], semg.at[b]).wait()
            pltpu.async_copy(rows.at[b], accS.at[idx_d.at[j]], sems.at[b], add=True)

            @pl.when(j + 2 < GRP)
            def _():
                pltpu.make_async_copy(rows.at[b], accS.at[idx_d.at[j]], sems.at[b]).wait()
                pltpu.async_copy(h_hbm.at[idx_s.at[j + 2]], rows.at[b], semg.at[b])

            return c2

        lax.fori_loop(0, GRP, chunk, carry)
        pltpu.make_async_copy(rows.at[0], accS.at[idx_d.at[0]], sems.at[0]).wait()
        pltpu.make_async_copy(rows.at[1], accS.at[idx_d.at[1]], sems.at[1]).wait()
        return carry

    lax.fori_loop(0, NGRP, group, 0)
    plsc.subcore_barrier()
    pltpu.sync_copy(accS.at[pl.ds(r0, RPT)], outS.at[cid, pl.ds(r0, RPT)])


_sc_spmm = pl.kernel(
    _sc_spmm_body,
    out_type=jax.ShapeDtypeStruct((NC, NP, D), jnp.float32),
    mesh=_mesh,
    compiler_params=_sc_params,
    scratch_types=[
        pltpu.VMEM((GRP, CH), jnp.int32),
        pltpu.VMEM((GRP, CH), jnp.int32),
        pltpu.VMEM((2, CH, D), jnp.float32),
        pltpu.VMEM_SHARED((NP, D), jnp.float32),
        pltpu.SemaphoreType.DMA((2,)),
        pltpu.SemaphoreType.DMA((2,)),
    ],
)


CHE = 125
GRP2 = 16
NGRP2 = 5


def _sc_edgestat_body(dst_hbm, ef_hbm, z16_hbm, ones_hbm,
                      outE, outD,
                      idx_d, efg, onesv, accE, accD, semE, semD):
    cid = lax.axis_index("c")
    sid = lax.axis_index("s")
    wid = sid * NC + cid
    r0 = sid * RPT
    pltpu.sync_copy(z16_hbm.at[pl.ds(r0, RPT)], accE.at[pl.ds(r0, RPT)])
    pltpu.sync_copy(z16_hbm.at[pl.ds(r0, RPT)], accD.at[pl.ds(r0, RPT)])
    pltpu.sync_copy(ones_hbm, onesv)
    plsc.subcore_barrier()

    def group(g, carry):
        pltpu.async_copy(dst_hbm.at[wid, g], idx_d, semE)
        pltpu.async_copy(ef_hbm.at[wid, g], efg, semD)
        pltpu.make_async_copy(dst_hbm.at[wid, g], idx_d, semE).wait()
        pltpu.make_async_copy(ef_hbm.at[wid, g], efg, semD).wait()

        def chunk(j, c2):
            pltpu.async_copy(efg.at[j], accE.at[idx_d.at[j]], semE, add=True)
            pltpu.async_copy(onesv, accD.at[idx_d.at[j]], semD, add=True)
            return c2

        lax.fori_loop(0, GRP2, chunk, carry)

        def drain(j, c2):
            pltpu.make_async_copy(efg.at[0], accE.at[idx_d.at[0]], semE).wait()
            pltpu.make_async_copy(onesv, accD.at[idx_d.at[0]], semD).wait()
            return c2

        lax.fori_loop(0, GRP2, drain, carry)
        return carry

    lax.fori_loop(0, NGRP2, group, 0)
    plsc.subcore_barrier()
    pltpu.sync_copy(accE.at[pl.ds(r0, RPT)], outE.at[cid, pl.ds(r0, RPT)])
    pltpu.sync_copy(accD.at[pl.ds(r0, RPT)], outD.at[cid, pl.ds(r0, RPT)])


_sc_edgestat = pl.kernel(
    _sc_edgestat_body,
    out_type=[
        jax.ShapeDtypeStruct((NC, NP, DE), jnp.float32),
        jax.ShapeDtypeStruct((NC, NP, DE), jnp.float32),
    ],
    mesh=_mesh,
    compiler_params=_sc_params,
    scratch_types=[
        pltpu.VMEM((GRP2, CHE), jnp.int32),
        pltpu.VMEM((GRP2, CHE, DE), jnp.float32),
        pltpu.VMEM((CHE, DE), jnp.float32),
        pltpu.VMEM_SHARED((NP, DE), jnp.float32),
        pltpu.VMEM_SHARED((NP, DE), jnp.float32),
        pltpu.SemaphoreType.DMA,
        pltpu.SemaphoreType.DMA,
    ],
)


def _sc_edge_body(s_hbm, src_hbm, dst_hbm,
                  out0_hbm, out1_hbm,
                  sv_v, dv_v, s_v, o0, o1, sem3):
    cid = lax.axis_index("c")
    sid = lax.axis_index("s")
    wid = sid * NC + cid
    e0 = wid * EPW
    pltpu.async_copy(s_hbm, s_v, sem3.at[0])
    pltpu.async_copy(src_hbm.at[pl.ds(e0, EPW)], sv_v, sem3.at[1])
    pltpu.async_copy(dst_hbm.at[pl.ds(e0, EPW)], dv_v, sem3.at[2])
    pltpu.make_async_copy(s_hbm, s_v, sem3.at[0]).wait()
    pltpu.make_async_copy(src_hbm.at[pl.ds(e0, EPW)], sv_v, sem3.at[1]).wait()
    pltpu.make_async_copy(dst_hbm.at[pl.ds(e0, EPW)], dv_v, sem3.at[2]).wait()

    def it(i, carry):
        sl = pl.ds(i * 16, 16)
        s4 = sv_v[sl] * 4
        d4 = dv_v[sl] * 4
        o0[sl] = plsc.load_gather(s_v, [s4]) + plsc.load_gather(s_v, [d4 + 2])
        o1[sl] = plsc.load_gather(s_v, [s4 + 1]) + plsc.load_gather(s_v, [d4 + 3])
        return carry

    lax.fori_loop(0, EPW // 16, it, 0)
    pltpu.sync_copy(o0, out0_hbm.at[pl.ds(e0, EPW)])
    pltpu.sync_copy(o1, out1_hbm.at[pl.ds(e0, EPW)])


_sc_edge = pl.kernel(
    _sc_edge_body,
    out_type=[
        jax.ShapeDtypeStruct((E,), jnp.float32),
        jax.ShapeDtypeStruct((E,), jnp.float32),
    ],
    mesh=_mesh,
    compiler_params=_sc_params_nl,
    scratch_types=[
        pltpu.VMEM((EPW,), jnp.int32),
        pltpu.VMEM((EPW,), jnp.int32),
        pltpu.VMEM((N * 4,), jnp.float32),
        pltpu.VMEM((EPW,), jnp.float32),
        pltpu.VMEM((EPW,), jnp.float32),
        pltpu.SemaphoreType.DMA((3,)),
    ],
)


BT = 2000
GRID = N // BT


def _tc_layer1_body(S2, h, E2, D2, Wm, bm, Wa, ba, out):
    Sv = S2[0] + S2[1]
    Ea = E2[0] + E2[1]
    deg = D2[0][:, 0:1] + D2[1][:, 0:1]
    hn = jnp.dot(Sv, Wm[0:D], preferred_element_type=jnp.float32)
    hn += jnp.dot(Ea, Wm[D:D + DE], preferred_element_type=jnp.float32)
    hn += deg * bm[...]
    hn /= jnp.maximum(deg, 1.0)
    o = jnp.dot(h[...], Wa[0:D], preferred_element_type=jnp.float32)
    o += jnp.dot(hn, Wa[D:2 * D], preferred_element_type=jnp.float32)
    out[...] = jnp.maximum(o + ba[...], 0.0)


def _tc_layer2_body(S2, h, E2, D2, Wm, bm, Wa, ba, W4, b4, outa):
    Sv = S2[0] + S2[1]
    Ea = E2[0] + E2[1]
    deg = D2[0][:, 0:1] + D2[1][:, 0:1]
    hn = jnp.dot(Sv, Wm[0:D], preferred_element_type=jnp.float32)
    hn += jnp.dot(Ea, Wm[D:D + DE], preferred_element_type=jnp.float32)
    hn += deg * bm[...]
    hn /= jnp.maximum(deg, 1.0)
    o = jnp.dot(h[...], Wa[0:D], preferred_element_type=jnp.float32)
    o += jnp.dot(hn, Wa[D:2 * D], preferred_element_type=jnp.float32)
    h2 = jnp.maximum(o + ba[...], 0.0)
    outa[...] = jnp.dot(h2, W4[...], preferred_element_type=jnp.float32) + b4[...]


def _row_spec(shape):
    nd = len(shape)
    if nd == 3:
        return pl.BlockSpec((shape[0], BT, shape[2]), lambda i: (0, i, 0))
    return pl.BlockSpec((BT, shape[1]), lambda i: (i, 0))


def _full_spec(shape):
    return pl.BlockSpec(shape, lambda i: tuple(0 for _ in shape))


def _tc_layer1(S2, h, E2, D2, Wm, bm, Wa, ba):
    return pl.pallas_call(
        _tc_layer1_body,
        grid=(GRID,),
        in_specs=[
            _row_spec((NC, NP, D)), _row_spec((N, D)),
            _row_spec((NC, NP, DE)), _row_spec((NC, NP, DE)),
            _full_spec((D + DE, D)), _full_spec((1, D)),
            _full_spec((2 * D, D)), _full_spec((1, D)),
        ],
        out_specs=_row_spec((N, D)),
        out_shape=jax.ShapeDtypeStruct((N, D), jnp.float32),
    )(S2, h, E2, D2, Wm, bm, Wa, ba)


def _tc_layer2(S2, h, E2, D2, Wm, bm, Wa, ba, W4, b4):
    return pl.pallas_call(
        _tc_layer2_body,
        grid=(GRID,),
        in_specs=[
            _row_spec((NC, NP, D)), _row_spec((N, D)),
            _row_spec((NC, NP, DE)), _row_spec((NC, NP, DE)),
            _full_spec((D + DE, D)), _full_spec((1, D)),
            _full_spec((2 * D, D)), _full_spec((1, D)),
            _full_spec((D, 4)), _full_spec((1, 4)),
        ],
        out_specs=_row_spec((N, 4)),
        out_shape=jax.ShapeDtypeStruct((N, 4), jnp.float32),
    )(S2, h, E2, D2, Wm, bm, Wa, ba, W4, b4)


def kernel(nfeats, efeats, edge_index, Wm1, bm1, Wa1, ba1, Wm2, bm2, Wa2, ba2, Wp, bp):
    h0 = nfeats[:, 0, :]
    ef = efeats[:, 0, :]
    src = edge_index[0]
    dst = edge_index[1]
    src4d = src.reshape(NW, NGRP, GRP, CH)
    dst4d = dst.reshape(NW, NGRP, GRP, CH)
    z128 = jnp.zeros((CH, D), jnp.float32)
    z16 = jnp.zeros((NP, DE), jnp.float32)
    ones = jnp.zeros((CH, DE), jnp.float32).at[:, 0].set(1.0)

    dst5d = dst.reshape(NW, NGRP2, GRP2, CHE)
    ef5d = ef.reshape(NW, NGRP2, GRP2, CHE, DE)
    ones125 = jnp.zeros((CHE, DE), jnp.float32).at[:, 0].set(1.0)
    E2, D2 = _sc_edgestat(dst5d, ef5d, z16, ones125)
    S2 = _sc_spmm(h0, src4d, dst4d, z128)
    h1 = _tc_layer1(S2, h0, E2, D2, Wm1, bm1.reshape(1, D), Wa1, ba1.reshape(1, D))
    S2b = _sc_spmm(h1, src4d, dst4d, z128)
    W4 = jnp.concatenate([Wp[:D], Wp[D:]], axis=1)
    b4 = (jnp.concatenate([bp, bp]) * 0.5).reshape(1, 4)
    s4 = _tc_layer2(S2b, h1, E2, D2, Wm2, bm2.reshape(1, D), Wa2, ba2.reshape(1, D),
                    W4, b4)
    o0, o1 = _sc_edge(s4.reshape(N * 4), src, dst)
    return jnp.stack([o0, o1], axis=1)
